# initial kernel scaffold (unmeasured)
import jax
import jax.numpy as jnp
from jax import lax
from jax.experimental import pallas as pl
from jax.experimental.pallas import tpu as pltpu

M_BLOCK = 4096
D = 4096
CHUNK = 512
K = M_BLOCK // CHUNK


def kernel(partial, gamma):
    def body(partial_ref, gamma_ref, out_ref,
             mine_v, theirs_f32_v, send_v, recv_v, outc_v,
             send_sems, recv_sems, sem_mine, sem_theirs, sem_out):
        my_x = lax.axis_index("x")
        my_y = lax.axis_index("y")
        peer_x = 1 - my_x

        barrier = pltpu.get_barrier_semaphore()
        pl.semaphore_signal(
            barrier, inc=1,
            device_id=(peer_x, my_y), device_id_type=pl.DeviceIdType.MESH,
        )
        pl.semaphore_wait(barrier, 1)

        g = gamma_ref[:]

        for k in range(K):
            slot = k % 2
            cp_t = pltpu.make_async_copy(
                partial_ref.at[0, pl.ds(peer_x * M_BLOCK + k * CHUNK, CHUNK), :],
                theirs_f32_v, sem_theirs)
            cp_t.start()
            cp_m = pltpu.make_async_copy(
                partial_ref.at[0, pl.ds(my_x * M_BLOCK + k * CHUNK, CHUNK), :],
                mine_v, sem_mine)
            cp_m.start()
            cp_t.wait()
            send_v[slot, :, :] = theirs_f32_v[:, :].astype(jnp.bfloat16)
            rdma = pltpu.make_async_remote_copy(
                src_ref=send_v.at[slot],
                dst_ref=recv_v.at[slot],
                send_sem=send_sems.at[slot],
                recv_sem=recv_sems.at[slot],
                device_id=(peer_x, my_y),
                device_id_type=pl.DeviceIdType.MESH,
            )
            rdma.start()
            rdma.wait()

            cp_m.wait()
            s = mine_v[:, :] + recv_v[slot, :, :].astype(jnp.float32)
            ms = jnp.mean(s * s, axis=-1, keepdims=True)
            outc_v[:, :] = s * lax.rsqrt(ms + 1e-6) * g

            cp_o = pltpu.make_async_copy(
                outc_v, out_ref.at[pl.ds(k * CHUNK, CHUNK), :], sem_out)
            cp_o.start()
            cp_o.wait()

    return pl.pallas_call(
        body,
        out_shape=jax.ShapeDtypeStruct((M_BLOCK, D), jnp.float32),
        in_specs=[
            pl.BlockSpec(memory_space=pltpu.ANY),
            pl.BlockSpec(memory_space=pltpu.VMEM),
        ],
        out_specs=pl.BlockSpec(memory_space=pltpu.ANY),
        scratch_shapes=[
            pltpu.VMEM((CHUNK, D), jnp.float32),
            pltpu.VMEM((CHUNK, D), jnp.float32),
            pltpu.VMEM((2, CHUNK, D), jnp.bfloat16),
            pltpu.VMEM((2, CHUNK, D), jnp.bfloat16),
            pltpu.VMEM((CHUNK, D), jnp.float32),
            pltpu.SemaphoreType.DMA((2,)),
            pltpu.SemaphoreType.DMA((2,)),
            pltpu.SemaphoreType.DMA,
            pltpu.SemaphoreType.DMA,
            pltpu.SemaphoreType.DMA,
        ],
        compiler_params=pltpu.CompilerParams(collective_id=0),
    )(partial, gamma)


# baseline (device time: 498769 ns/iter reference)
import jax
import jax.numpy as jnp
from jax import lax
from jax.experimental import pallas as pl
from jax.experimental.pallas import tpu as pltpu

M_BLOCK = 4096
D = 4096
CHUNK = 512
K = M_BLOCK // CHUNK


def kernel(partial, gamma):
    def body(partial_ref, gamma_ref, out_ref,
             mine_v, theirs_f32_v, send_v, recv_v, outc_v,
             send_sems, recv_sems, sem_mine, sem_theirs, sem_out):
        my_x = lax.axis_index("x")
        my_y = lax.axis_index("y")
        peer_x = 1 - my_x

        barrier = pltpu.get_barrier_semaphore()
        pl.semaphore_signal(
            barrier, inc=1,
            device_id=(peer_x, my_y), device_id_type=pl.DeviceIdType.MESH,
        )
        pl.semaphore_wait(barrier, 1)

        g = gamma_ref[:]

        for k in range(K):
            slot = k % 2
            cp_t = pltpu.make_async_copy(
                partial_ref.at[0, pl.ds(peer_x * M_BLOCK + k * CHUNK, CHUNK), :],
                theirs_f32_v, sem_theirs)
            cp_t.start()
            cp_m = pltpu.make_async_copy(
                partial_ref.at[0, pl.ds(my_x * M_BLOCK + k * CHUNK, CHUNK), :],
                mine_v, sem_mine)
            cp_m.start()
            cp_t.wait()
            send_v[slot, :, :] = theirs_f32_v[:, :].astype(jnp.bfloat16)
            rdma = pltpu.make_async_remote_copy(
                src_ref=send_v.at[slot],
                dst_ref=recv_v.at[slot],
                send_sem=send_sems.at[slot],
                recv_sem=recv_sems.at[slot],
                device_id=(peer_x, my_y),
                device_id_type=pl.DeviceIdType.MESH,
            )
            rdma.start()
            rdma.wait()

            cp_m.wait()
            s = mine_v[:, :] + recv_v[slot, :, :].astype(jnp.float32)
            ms = jnp.mean(s * s, axis=-1, keepdims=True)
            outc_v[:, :] = s * lax.rsqrt(ms + 1e-6) * g

            cp_o = pltpu.make_async_copy(
                outc_v, out_ref.at[pl.ds(k * CHUNK, CHUNK), :], sem_out)
            cp_o.start()
            cp_o.wait()

    return pl.pallas_call(
        body,
        out_shape=jax.ShapeDtypeStruct((M_BLOCK, D), jnp.float32),
        in_specs=[
            pl.BlockSpec(memory_space=pl.ANY),
            pl.BlockSpec(memory_space=pltpu.MemorySpace.VMEM),
        ],
        out_specs=pl.BlockSpec(memory_space=pl.ANY),
        scratch_shapes=[
            pltpu.VMEM((CHUNK, D), jnp.float32),
            pltpu.VMEM((CHUNK, D), jnp.float32),
            pltpu.VMEM((2, CHUNK, D), jnp.bfloat16),
            pltpu.VMEM((2, CHUNK, D), jnp.bfloat16),
            pltpu.VMEM((CHUNK, D), jnp.float32),
            pltpu.SemaphoreType.DMA((2,)),
            pltpu.SemaphoreType.DMA((2,)),
            pltpu.SemaphoreType.DMA,
            pltpu.SemaphoreType.DMA,
            pltpu.SemaphoreType.DMA,
        ],
        compiler_params=pltpu.CompilerParams(
            collective_id=0, vmem_limit_bytes=100 * 1024 * 1024
        ),
    )(partial, gamma)


# device time: 286730 ns/iter; 1.7395x vs baseline; 1.7395x over previous
import jax
import jax.numpy as jnp
from jax import lax
from jax.experimental import pallas as pl
from jax.experimental.pallas import tpu as pltpu

M_BLOCK = 4096
D = 4096
HALF = M_BLOCK // 2
C = 128
KH = HALF // C


def kernel(partial, gamma):
    def body(partial_ref, gamma_ref, out_ref,
             mine_v, theirs_v, xsend, xrecv, ysend, yrecv, outc,
             sem_mine, sem_theirs, sx, rx, sy, ry, so):
        my_x = lax.axis_index("x")
        my_y = lax.axis_index("y")
        peer_x = 1 - my_x
        peer_y = 1 - my_y

        barrier = pltpu.get_barrier_semaphore()
        pl.semaphore_signal(
            barrier, inc=1,
            device_id=(peer_x, my_y), device_id_type=pl.DeviceIdType.MESH,
        )
        pl.semaphore_signal(
            barrier, inc=1,
            device_id=(my_x, peer_y), device_id_type=pl.DeviceIdType.MESH,
        )
        pl.semaphore_wait(barrier, 2)

        g = gamma_ref[:]
        mine_base = my_x * M_BLOCK + my_y * HALF
        theirs_base = peer_x * M_BLOCK + my_y * HALF

        def load(base, k, buf, sems):
            cp = pltpu.make_async_copy(
                partial_ref.at[0, pl.ds(base + k * C, C), :],
                buf.at[k % 2], sems.at[k % 2])
            cp.start()
            return cp

        pending = {0: None, 1: None}

        def store_out(row_start, slot, value):
            if pending[slot] is not None:
                pending[slot].wait()
            outc[slot, :, :] = value
            cp = pltpu.make_async_copy(
                outc.at[slot], out_ref.at[pl.ds(row_start, C), :], so.at[slot])
            cp.start()
            pending[slot] = cp

        rdmas_x = []
        rdmas_y = []
        cps_m = {0: load(mine_base, 0, mine_v, sem_mine)}
        cps_t = {0: load(theirs_base, 0, theirs_v, sem_theirs)}

        for k in range(KH):
            if k + 1 < KH:
                cps_t[k + 1] = load(theirs_base, k + 1, theirs_v, sem_theirs)
                cps_m[k + 1] = load(mine_base, k + 1, mine_v, sem_mine)
            cps_t[k].wait()
            if k >= 2:
                rdmas_x[k - 2].wait_send()
            xsend[k % 2, :, :] = theirs_v[k % 2, :, :].astype(jnp.bfloat16)
            rdma_x = pltpu.make_async_remote_copy(
                src_ref=xsend.at[k % 2],
                dst_ref=xrecv.at[pl.ds(k * C, C), :],
                send_sem=sx.at[k],
                recv_sem=rx.at[k],
                device_id=(peer_x, my_y),
                device_id_type=pl.DeviceIdType.MESH,
            )
            rdma_x.start()
            rdmas_x.append(rdma_x)
            cps_m[k].wait()
            rdma_x.wait_recv()
            s = mine_v[k % 2, :, :] + xrecv[pl.ds(k * C, C), :].astype(jnp.float32)
            ms = jnp.mean(s * s, axis=-1, keepdims=True)
            o = s * lax.rsqrt(ms + 1e-6) * g
            store_out(my_y * HALF + k * C, k % 2, o)
            if k >= 2:
                rdmas_y[k - 2].wait_send()
            ysend[k % 2, :, :] = o.astype(jnp.bfloat16)
            rdma_y = pltpu.make_async_remote_copy(
                src_ref=ysend.at[k % 2],
                dst_ref=yrecv.at[pl.ds(k * C, C), :],
                send_sem=sy.at[k],
                recv_sem=ry.at[k],
                device_id=(my_x, peer_y),
                device_id_type=pl.DeviceIdType.MESH,
            )
            rdma_y.start()
            rdmas_y.append(rdma_y)

        for k in range(KH):
            rdmas_y[k].wait_recv()
            store_out(peer_y * HALF + k * C, k % 2,
                      yrecv[pl.ds(k * C, C), :].astype(jnp.float32))

        for k in range(max(0, KH - 2), KH):
            rdmas_x[k].wait_send()
            rdmas_y[k].wait_send()
        pending[0].wait()
        pending[1].wait()

    return pl.pallas_call(
        body,
        out_shape=jax.ShapeDtypeStruct((M_BLOCK, D), jnp.float32),
        in_specs=[
            pl.BlockSpec(memory_space=pl.ANY),
            pl.BlockSpec(memory_space=pltpu.MemorySpace.VMEM),
        ],
        out_specs=pl.BlockSpec(memory_space=pl.ANY),
        scratch_shapes=[
            pltpu.VMEM((2, C, D), jnp.float32),
            pltpu.VMEM((2, C, D), jnp.float32),
            pltpu.VMEM((2, C, D), jnp.bfloat16),
            pltpu.VMEM((HALF, D), jnp.bfloat16),
            pltpu.VMEM((2, C, D), jnp.bfloat16),
            pltpu.VMEM((HALF, D), jnp.bfloat16),
            pltpu.VMEM((2, C, D), jnp.float32),
            pltpu.SemaphoreType.DMA((2,)),
            pltpu.SemaphoreType.DMA((2,)),
            pltpu.SemaphoreType.DMA((KH,)),
            pltpu.SemaphoreType.DMA((KH,)),
            pltpu.SemaphoreType.DMA((KH,)),
            pltpu.SemaphoreType.DMA((KH,)),
            pltpu.SemaphoreType.DMA((2,)),
        ],
        compiler_params=pltpu.CompilerParams(
            collective_id=0, vmem_limit_bytes=100 * 1024 * 1024
        ),
    )(partial, gamma)


# device time: 246086 ns/iter; 2.0268x vs baseline; 1.1652x over previous
import jax
import jax.numpy as jnp
from jax import lax
from jax.experimental import pallas as pl
from jax.experimental.pallas import tpu as pltpu

M_BLOCK = 4096
D = 4096
HALF = M_BLOCK // 2
C = 128
KH = HALF // C
LEAD = 3
NSEND = 4


def kernel(partial, gamma):
    def body(partial_ref, gamma_ref, out_ref,
             mine_v, theirs_v, xsend, xrecv, ysend, yrecv, outc,
             sem_mine, sem_theirs, sx, rx, sy, ry, so):
        my_x = lax.axis_index("x")
        my_y = lax.axis_index("y")
        peer_x = 1 - my_x
        peer_y = 1 - my_y

        barrier = pltpu.get_barrier_semaphore()
        pl.semaphore_signal(
            barrier, inc=1,
            device_id=(peer_x, my_y), device_id_type=pl.DeviceIdType.MESH,
        )
        pl.semaphore_signal(
            barrier, inc=1,
            device_id=(my_x, peer_y), device_id_type=pl.DeviceIdType.MESH,
        )
        pl.semaphore_wait(barrier, 2)

        g = gamma_ref[:]
        mine_base = my_x * M_BLOCK + my_y * HALF
        theirs_base = peer_x * M_BLOCK + my_y * HALF

        def load(base, k, buf, sems):
            cp = pltpu.make_async_copy(
                partial_ref.at[0, pl.ds(base + k * C, C), :],
                buf.at[k % 2], sems.at[k % 2])
            cp.start()
            return cp

        pending = {0: None, 1: None}

        def store_out(row_start, slot, value):
            if pending[slot] is not None:
                pending[slot].wait()
            outc[slot, :, :] = value
            cp = pltpu.make_async_copy(
                outc.at[slot], out_ref.at[pl.ds(row_start, C), :], so.at[slot])
            cp.start()
            pending[slot] = cp

        def drain_y(j):
            rdmas_y[j].wait_recv()
            store_out(peer_y * HALF + j * C, j % 2,
                      yrecv[pl.ds(j * C, C), :].astype(jnp.float32))

        rdmas_x = [None] * KH
        rdmas_y = [None] * KH
        cps_m = {0: load(mine_base, 0, mine_v, sem_mine)}
        cps_t = {0: load(theirs_base, 0, theirs_v, sem_theirs)}

        for k in range(KH + LEAD):
            if k < KH:
                if k + 1 < KH:
                    cps_t[k + 1] = load(theirs_base, k + 1, theirs_v, sem_theirs)
                cps_t[k].wait()
                if k >= NSEND:
                    rdmas_x[k - NSEND].wait_send()
                xsend[k % NSEND, :, :] = theirs_v[k % 2, :, :].astype(jnp.bfloat16)
                rdma_x = pltpu.make_async_remote_copy(
                    src_ref=xsend.at[k % NSEND],
                    dst_ref=xrecv.at[pl.ds(k * C, C), :],
                    send_sem=sx.at[k % NSEND],
                    recv_sem=rx.at[k],
                    device_id=(peer_x, my_y),
                    device_id_type=pl.DeviceIdType.MESH,
                )
                rdma_x.start()
                rdmas_x[k] = rdma_x
            j = k - LEAD
            if j >= 0:
                if j + 1 < KH:
                    cps_m[j + 1] = load(mine_base, j + 1, mine_v, sem_mine)
                cps_m[j].wait()
                rdmas_x[j].wait_recv()
                s = (mine_v[j % 2, :, :]
                     + xrecv[pl.ds(j * C, C), :].astype(jnp.float32))
                ms = jnp.mean(s * s, axis=-1, keepdims=True)
                o = s * lax.rsqrt(ms + 1e-6) * g
                store_out(my_y * HALF + j * C, j % 2, o)
                if j >= NSEND:
                    rdmas_y[j - NSEND].wait_send()
                ysend[j % NSEND, :, :] = o.astype(jnp.bfloat16)
                rdma_y = pltpu.make_async_remote_copy(
                    src_ref=ysend.at[j % NSEND],
                    dst_ref=yrecv.at[pl.ds(j * C, C), :],
                    send_sem=sy.at[j % NSEND],
                    recv_sem=ry.at[j],
                    device_id=(my_x, peer_y),
                    device_id_type=pl.DeviceIdType.MESH,
                )
                rdma_y.start()
                rdmas_y[j] = rdma_y
                if j >= 2:
                    drain_y(j - 2)

        for j in range(KH - 2, KH):
            drain_y(j)

        for k in range(KH - NSEND, KH):
            rdmas_x[k].wait_send()
            rdmas_y[k].wait_send()
        pending[0].wait()
        pending[1].wait()

    return pl.pallas_call(
        body,
        out_shape=jax.ShapeDtypeStruct((M_BLOCK, D), jnp.float32),
        in_specs=[
            pl.BlockSpec(memory_space=pl.ANY),
            pl.BlockSpec(memory_space=pltpu.MemorySpace.VMEM),
        ],
        out_specs=pl.BlockSpec(memory_space=pl.ANY),
        scratch_shapes=[
            pltpu.VMEM((2, C, D), jnp.float32),
            pltpu.VMEM((2, C, D), jnp.float32),
            pltpu.VMEM((NSEND, C, D), jnp.bfloat16),
            pltpu.VMEM((HALF, D), jnp.bfloat16),
            pltpu.VMEM((NSEND, C, D), jnp.bfloat16),
            pltpu.VMEM((HALF, D), jnp.bfloat16),
            pltpu.VMEM((2, C, D), jnp.float32),
            pltpu.SemaphoreType.DMA((2,)),
            pltpu.SemaphoreType.DMA((2,)),
            pltpu.SemaphoreType.DMA((NSEND,)),
            pltpu.SemaphoreType.DMA((KH,)),
            pltpu.SemaphoreType.DMA((NSEND,)),
            pltpu.SemaphoreType.DMA((KH,)),
            pltpu.SemaphoreType.DMA((2,)),
        ],
        compiler_params=pltpu.CompilerParams(
            collective_id=0, vmem_limit_bytes=100 * 1024 * 1024
        ),
    )(partial, gamma)
